# Initial kernel scaffold; baseline (speedup 1.0000x reference)
import jax
import jax.numpy as jnp
from jax.experimental import pallas as pl


def kernel(codes, code_embed_weight):
    raise NotImplementedError("write your pallas kernel here")


# speedup vs baseline: 13.6067x; 13.6067x over previous
import functools

import jax
import jax.numpy as jnp
from jax import lax
from jax.experimental import pallas as pl
from jax.experimental.pallas import tpu as pltpu
from jax.experimental.pallas import tpu_sc as plsc

_B, _T, _K, _D = 64, 4096, 512, 8
_NC, _NS, _L = 2, 16, 16
_NW = _NC * _NS
_BPW = _B // _NW


def _nn_body(w_ref, nn_ref):
    w = w_ref[...]
    w2_col = jnp.sum(w * w, axis=1, keepdims=True)
    ones = jnp.ones((1, _D), jnp.float32)
    w2_row = lax.dot_general(ones, w * w,
                             (((1,), (1,)), ((), ())),
                             preferred_element_type=jnp.float32)
    g = lax.dot_general(w, w, (((1,), (1,)), ((), ())),
                        preferred_element_type=jnp.float32)
    d2 = jnp.maximum(w2_col - 2.0 * g + w2_row, 0.0)
    dist = jnp.sqrt(d2)
    mn = jnp.min(dist, axis=1, keepdims=True)
    col = lax.broadcasted_iota(jnp.int32, (_K, _K), 1)
    nn = jnp.min(jnp.where(dist == mn, col, _K), axis=1, keepdims=True)
    nn_ref[...] = nn


_nn_call = pl.pallas_call(
    _nn_body,
    out_shape=jax.ShapeDtypeStruct((_K, 1), jnp.int32),
)


_sc_mesh = plsc.VectorSubcoreMesh(core_axis_name="c", subcore_axis_name="s")


@functools.partial(
    pl.kernel,
    mesh=_sc_mesh,
    compiler_params=pltpu.CompilerParams(needs_layout_passes=False),
    out_type=[
        jax.ShapeDtypeStruct((_B, _D, _T), jnp.float32),
        jax.ShapeDtypeStruct((_B, _T), jnp.int32),
    ],
    scratch_types=[
        pltpu.VMEM((_K, _D), jnp.float32),
        pltpu.VMEM((_K,), jnp.int32),
        pltpu.VMEM((_T,), jnp.int32),
        pltpu.VMEM((_D, _T), jnp.float32),
        pltpu.VMEM((_T,), jnp.int32),
    ],
)
def _sc_gather(w_hbm, nn_hbm, codes_hbm, lat_hbm, ch_hbm,
               w_v, nn_v, codes_v, lat_v, ch_v):
    wid = lax.axis_index("s") * _NC + lax.axis_index("c")
    pltpu.sync_copy(w_hbm, w_v)
    pltpu.sync_copy(nn_hbm, nn_v)
    for i in range(_BPW):
        b = wid * _BPW + i
        pltpu.sync_copy(codes_hbm.at[b], codes_v)

        def step(t, carry):
            idx = codes_v[pl.ds(t * _L, _L)]
            ch_v[pl.ds(t * _L, _L)] = plsc.load_gather(nn_v, [idx])
            for d in range(_D):
                dvec = jnp.full((_L,), d, jnp.int32)
                lat_v[d, pl.ds(t * _L, _L)] = plsc.load_gather(w_v, [idx, dvec])
            return carry

        lax.fori_loop(0, _T // _L, step, 0)
        pltpu.sync_copy(lat_v, lat_hbm.at[b])
        pltpu.sync_copy(ch_v, ch_hbm.at[b])


def kernel(codes, code_embed_weight):
    codes = codes.astype(jnp.int32)
    w = code_embed_weight.astype(jnp.float32)
    nn = _nn_call(w).reshape(_K)
    latents, codes_hat = _sc_gather(w, nn, codes)
    return latents, codes_hat


# speedup vs baseline: 17.2011x; 1.2642x over previous
import functools

import jax
import jax.numpy as jnp
from jax import lax
from jax.experimental import pallas as pl
from jax.experimental.pallas import tpu as pltpu
from jax.experimental.pallas import tpu_sc as plsc

_B, _T, _K, _D = 64, 4096, 512, 8
_NC, _NS, _L = 2, 16, 16
_NW = _NC * _NS
_BPW = _B // _NW


def _nn_body(w_ref, nn_ref):
    w = w_ref[...]
    w2_col = jnp.sum(w * w, axis=1, keepdims=True)
    ones = jnp.ones((1, _D), jnp.float32)
    w2_row = lax.dot_general(ones, w * w,
                             (((1,), (1,)), ((), ())),
                             preferred_element_type=jnp.float32)
    g = lax.dot_general(w, w, (((1,), (1,)), ((), ())),
                        preferred_element_type=jnp.float32)
    d2 = jnp.maximum(w2_col - 2.0 * g + w2_row, 0.0)
    dist = jnp.sqrt(d2)
    mn = jnp.min(dist, axis=1, keepdims=True)
    col = lax.broadcasted_iota(jnp.int32, (_K, _K), 1)
    nn = jnp.min(jnp.where(dist == mn, col, _K), axis=1, keepdims=True)
    nn_ref[...] = nn


_nn_call = pl.pallas_call(
    _nn_body,
    out_shape=jax.ShapeDtypeStruct((_K, 1), jnp.int32),
)


_sc_mesh = plsc.VectorSubcoreMesh(core_axis_name="c", subcore_axis_name="s")


@functools.partial(
    pl.kernel,
    mesh=_sc_mesh,
    compiler_params=pltpu.CompilerParams(needs_layout_passes=False),
    out_type=[
        jax.ShapeDtypeStruct((_B, _D, _T), jnp.float32),
        jax.ShapeDtypeStruct((_B, _T), jnp.int32),
    ],
    scratch_types=[
        pltpu.VMEM((_K, _D), jnp.float32),
        pltpu.VMEM((_K,), jnp.int32),
        pltpu.VMEM((_T,), jnp.int32),
        pltpu.VMEM((_D, _T), jnp.float32),
        pltpu.VMEM((_T,), jnp.int32),
    ],
)
def _sc_gather(w_hbm, nn_hbm, codes_hbm, lat_hbm, ch_hbm,
               w_v, nn_v, codes_v, lat_v, ch_v):
    wid = lax.axis_index("s") * _NC + lax.axis_index("c")
    pltpu.sync_copy(w_hbm, w_v)
    pltpu.sync_copy(nn_hbm, nn_v)
    for i in range(_BPW):
        b = wid * _BPW + i
        pltpu.sync_copy(codes_hbm.at[b], codes_v)

        @plsc.parallel_loop(0, _T, step=_L, unroll=8)
        def step(t):
            idx = codes_v[pl.ds(t, _L)]
            ch_v[pl.ds(t, _L)] = plsc.load_gather(nn_v, [idx])
            for d in range(_D):
                dvec = jnp.full((_L,), d, jnp.int32)
                lat_v[d, pl.ds(t, _L)] = plsc.load_gather(w_v, [idx, dvec])

        pltpu.sync_copy(lat_v, lat_hbm.at[b])
        pltpu.sync_copy(ch_v, ch_hbm.at[b])


def kernel(codes, code_embed_weight):
    codes = codes.astype(jnp.int32)
    w = code_embed_weight.astype(jnp.float32)
    nn = _nn_call(w).reshape(_K)
    latents, codes_hat = _sc_gather(w, nn, codes)
    return latents, codes_hat


# speedup vs baseline: 30.0579x; 1.7474x over previous
import functools

import jax
import jax.numpy as jnp
from jax import lax
from jax.experimental import pallas as pl
from jax.experimental.pallas import tpu as pltpu
from jax.experimental.pallas import tpu_sc as plsc

_B, _T, _K, _D = 64, 4096, 512, 8
_NC, _NS, _L = 2, 16, 16
_NW = _NC * _NS
_BPW = _B // _NW


def _nn_body(w_ref, nn_ref):
    w = w_ref[...]
    w2_col = jnp.sum(w * w, axis=1, keepdims=True)
    ones = jnp.ones((1, _D), jnp.float32)
    w2_row = lax.dot_general(ones, w * w,
                             (((1,), (1,)), ((), ())),
                             preferred_element_type=jnp.float32)
    g = lax.dot_general(w, w, (((1,), (1,)), ((), ())),
                        preferred_element_type=jnp.float32)
    d2 = jnp.maximum(w2_col - 2.0 * g + w2_row, 0.0)
    dist = jnp.sqrt(d2)
    mn = jnp.min(dist, axis=1, keepdims=True)
    col = lax.broadcasted_iota(jnp.int32, (_K, _K), 1)
    nn = jnp.min(jnp.where(dist == mn, col, _K), axis=1, keepdims=True)
    nn_ref[...] = nn


_nn_call = pl.pallas_call(
    _nn_body,
    out_shape=jax.ShapeDtypeStruct((_K, 1), jnp.int32),
)


_sc_mesh = plsc.VectorSubcoreMesh(core_axis_name="c", subcore_axis_name="s")


@functools.partial(
    pl.kernel,
    mesh=_sc_mesh,
    compiler_params=pltpu.CompilerParams(needs_layout_passes=False,
                                         disable_bounds_checks=True),
    out_type=[
        jax.ShapeDtypeStruct((_B, _D, _T), jnp.float32),
        jax.ShapeDtypeStruct((_B, _T), jnp.int32),
    ],
    scratch_types=[
        pltpu.VMEM((_K * _D,), jnp.float32),
        pltpu.VMEM((_K,), jnp.int32),
        pltpu.VMEM((_T,), jnp.int32),
        pltpu.VMEM((_D, _T), jnp.float32),
        pltpu.VMEM((_T,), jnp.int32),
    ],
)
def _sc_gather(w_hbm, nn_hbm, codes_hbm, lat_hbm, ch_hbm,
               w_v, nn_v, codes_v, lat_v, ch_v):
    wid = lax.axis_index("s") * _NC + lax.axis_index("c")
    pltpu.sync_copy(w_hbm, w_v)
    pltpu.sync_copy(nn_hbm, nn_v)
    for i in range(_BPW):
        b = wid * _BPW + i
        pltpu.sync_copy(codes_hbm.at[b], codes_v)

        @plsc.parallel_loop(0, _T, step=_L, unroll=8)
        def step(t):
            idx = codes_v[pl.ds(t, _L)]
            ch_v[pl.ds(t, _L)] = plsc.load_gather(nn_v, [idx])
            base = idx * _D
            for d in range(_D):
                lat_v[d, pl.ds(t, _L)] = plsc.load_gather(w_v, [base + d])

        pltpu.sync_copy(lat_v, lat_hbm.at[b])
        pltpu.sync_copy(ch_v, ch_hbm.at[b])


def kernel(codes, code_embed_weight):
    codes = codes.astype(jnp.int32)
    w = code_embed_weight.astype(jnp.float32)
    nn = _nn_call(w).reshape(_K)
    latents, codes_hat = _sc_gather(w.reshape(_K * _D), nn, codes)
    return latents, codes_hat


# speedup vs baseline: 32.0035x; 1.0647x over previous
import functools

import jax
import jax.numpy as jnp
from jax import lax
from jax.experimental import pallas as pl
from jax.experimental.pallas import tpu as pltpu
from jax.experimental.pallas import tpu_sc as plsc

_B, _T, _K, _D = 64, 4096, 512, 8
_NC, _NS, _L = 2, 16, 16
_NW = _NC * _NS
_BPW = _B // _NW


def _nn_body(w_ref, nn_ref):
    w = w_ref[...]
    w2_col = jnp.sum(w * w, axis=1, keepdims=True)
    ones = jnp.ones((1, _D), jnp.float32)
    w2_row = lax.dot_general(ones, w * w,
                             (((1,), (1,)), ((), ())),
                             preferred_element_type=jnp.float32)
    g = lax.dot_general(w, w, (((1,), (1,)), ((), ())),
                        preferred_element_type=jnp.float32)
    d2 = jnp.maximum(w2_col - 2.0 * g + w2_row, 0.0)
    dist = jnp.sqrt(d2)
    mn = jnp.min(dist, axis=1, keepdims=True)
    col = lax.broadcasted_iota(jnp.int32, (_K, _K), 1)
    nn = jnp.min(jnp.where(dist == mn, col, _K), axis=1, keepdims=True)
    nn_ref[...] = nn


_nn_call = pl.pallas_call(
    _nn_body,
    out_shape=jax.ShapeDtypeStruct((_K, 1), jnp.int32),
)


_sc_mesh = plsc.VectorSubcoreMesh(core_axis_name="c", subcore_axis_name="s")


@functools.partial(
    pl.kernel,
    mesh=_sc_mesh,
    compiler_params=pltpu.CompilerParams(needs_layout_passes=False,
                                         disable_bounds_checks=True),
    out_type=[
        jax.ShapeDtypeStruct((_B, _D, _T), jnp.float32),
        jax.ShapeDtypeStruct((_B, _T), jnp.int32),
    ],
    scratch_types=[
        pltpu.VMEM((_K * _D,), jnp.float32),
        pltpu.VMEM((_K,), jnp.int32),
        pltpu.VMEM((_BPW, _T), jnp.int32),
        pltpu.VMEM((_BPW, _D, _T), jnp.float32),
        pltpu.VMEM((_BPW, _T), jnp.int32),
        pltpu.SemaphoreType.DMA,
        pltpu.SemaphoreType.DMA,
        pltpu.SemaphoreType.DMA,
    ],
)
def _sc_gather(w_hbm, nn_hbm, codes_hbm, lat_hbm, ch_hbm,
               w_v, nn_v, codes_v, lat_v, ch_v, sem_c0, sem_c1, sem_o):
    wid = lax.axis_index("s") * _NC + lax.axis_index("c")
    b0 = wid * _BPW
    sems = [sem_c0, sem_c1]
    in_cps = [pltpu.async_copy(codes_hbm.at[b0 + i], codes_v.at[i], sems[i])
              for i in range(_BPW)]
    pltpu.sync_copy(w_hbm, w_v)
    pltpu.sync_copy(nn_hbm, nn_v)
    out_cps = []
    for i in range(_BPW):
        in_cps[i].wait()

        @plsc.parallel_loop(0, _T, step=_L, unroll=8)
        def step(t):
            idx = codes_v[i, pl.ds(t, _L)]
            ch_v[i, pl.ds(t, _L)] = plsc.load_gather(nn_v, [idx])
            base = idx * _D
            for d in range(_D):
                lat_v[i, d, pl.ds(t, _L)] = plsc.load_gather(w_v, [base + d])

        out_cps.append(pltpu.async_copy(lat_v.at[i], lat_hbm.at[b0 + i], sem_o))
        out_cps.append(pltpu.async_copy(ch_v.at[i], ch_hbm.at[b0 + i], sem_o))
    for cp in out_cps:
        cp.wait()


def kernel(codes, code_embed_weight):
    codes = codes.astype(jnp.int32)
    w = code_embed_weight.astype(jnp.float32)
    nn = _nn_call(w).reshape(_K)
    latents, codes_hat = _sc_gather(w.reshape(_K * _D), nn, codes)
    return latents, codes_hat
